# baseline (speedup 1.0000x reference)
import jax
import jax.numpy as jnp
from jax import lax
from jax.experimental import pallas as pl
from jax.experimental.pallas import tpu as pltpu
from jax.experimental.pallas import tpu_sc as plsc

NUM_POSES = 100000
BATCH = 4096

_INFO = plsc.get_sparse_core_info()
_NC = _INFO.num_cores
_NS = _INFO.num_subcores
_NW = _NC * _NS
_BPW = BATCH // _NW
_L = 16
_QW = _BPW * 4
_TW = _BPW * 3


def _gather_body(idx_hbm, qflat_hbm, tflat_hbm, qo_hbm, to_hbm,
                 iv, qiv, tiv, qrows, trows, sem):
    wid = lax.axis_index("s") * _NC + lax.axis_index("c")
    base = wid * _BPW
    pltpu.sync_copy(idx_hbm.at[pl.ds(base, _BPW)], iv)

    lanes = jnp.arange(_L, dtype=jnp.int32)
    for c in range(_QW // _L):
        p = c * _L + lanes
        src = plsc.load_gather(iv, [p >> 2])
        qiv[pl.ds(c * _L, _L)] = src * 4 + (p & 3)
    for c in range(_TW // _L):
        p = c * _L + lanes
        d = (p * 21846) >> 16
        src = plsc.load_gather(iv, [d])
        tiv[pl.ds(c * _L, _L)] = src * 3 + (p - d * 3)

    cps = []
    for k in range(_QW // _BPW):
        cps.append(pltpu.async_copy(
            qflat_hbm.at[qiv.at[pl.ds(k * _BPW, _BPW)]],
            qrows.at[pl.ds(k * _BPW, _BPW)], sem))
    for k in range(_TW // _BPW):
        cps.append(pltpu.async_copy(
            tflat_hbm.at[tiv.at[pl.ds(k * _BPW, _BPW)]],
            trows.at[pl.ds(k * _BPW, _BPW)], sem))
    for cp in cps:
        cp.wait()

    pltpu.sync_copy(qrows, qo_hbm.at[pl.ds(base * 4, _QW)])
    pltpu.sync_copy(trows, to_hbm.at[pl.ds(base * 3, _TW)])


@jax.jit
def kernel(camera_pose_indices, q_pointcloud_camera_table, t_pointcloud_camera_table):
    idx = camera_pose_indices.astype(jnp.int32)
    gather = pl.kernel(
        _gather_body,
        out_type=(
            jax.ShapeDtypeStruct((BATCH * 4,), jnp.float32),
            jax.ShapeDtypeStruct((BATCH * 3,), jnp.float32),
        ),
        mesh=plsc.VectorSubcoreMesh(core_axis_name="c", subcore_axis_name="s"),
        scratch_types=[
            pltpu.VMEM((_BPW,), jnp.int32),
            pltpu.VMEM((_QW,), jnp.int32),
            pltpu.VMEM((_TW,), jnp.int32),
            pltpu.VMEM((_QW,), jnp.float32),
            pltpu.VMEM((_TW,), jnp.float32),
            pltpu.SemaphoreType.DMA,
        ],
        compiler_params=pltpu.CompilerParams(needs_layout_passes=False),
    )
    qo, to = gather(idx,
                    q_pointcloud_camera_table.reshape(-1),
                    t_pointcloud_camera_table.reshape(-1))
    return qo.reshape(BATCH, 4), to.reshape(BATCH, 3)


# speedup vs baseline: 1.0017x; 1.0017x over previous
import jax
import jax.numpy as jnp
from jax import lax
from jax.experimental import pallas as pl
from jax.experimental.pallas import tpu as pltpu
from jax.experimental.pallas import tpu_sc as plsc

NUM_POSES = 100000
BATCH = 4096

_INFO = plsc.get_sparse_core_info()
_NC = _INFO.num_cores
_NS = _INFO.num_subcores
_NW = _NC * _NS
_BPW = BATCH // _NW
_L = 16
_QW = _BPW * 4
_TW = _BPW * 3


def _gather_body(idx_hbm, qflat_hbm, tflat_hbm, qo_hbm, to_hbm,
                 iv, qiv, tiv, qrows, trows, sem):
    wid = lax.axis_index("s") * _NC + lax.axis_index("c")
    base = wid * _BPW
    pltpu.sync_copy(idx_hbm.at[pl.ds(base, _BPW)], iv)

    lanes = jnp.arange(_L, dtype=jnp.int32)
    for c in range(_QW // _L):
        p = c * _L + lanes
        src = plsc.load_gather(iv, [p >> 2])
        qiv[pl.ds(c * _L, _L)] = src * 4 + (p & 3)
    for c in range(_TW // _L):
        p = c * _L + lanes
        d = (p * 21846) >> 16
        src = plsc.load_gather(iv, [d])
        tiv[pl.ds(c * _L, _L)] = src * 3 + (p - d * 3)

    cps = []
    for k in range(_QW // _BPW):
        cps.append(pltpu.async_copy(
            qflat_hbm.at[qiv.at[pl.ds(k * _BPW, _BPW)]],
            qrows.at[pl.ds(k * _BPW, _BPW)], sem))
    for k in range(_TW // _BPW):
        cps.append(pltpu.async_copy(
            tflat_hbm.at[tiv.at[pl.ds(k * _BPW, _BPW)]],
            trows.at[pl.ds(k * _BPW, _BPW)], sem))
    for cp in cps:
        cp.wait()

    pltpu.sync_copy(qrows, qo_hbm.at[pl.ds(base * 4, _QW)])
    pltpu.sync_copy(trows, to_hbm.at[pl.ds(base * 3, _TW)])


@jax.jit
def kernel(camera_pose_indices, q_pointcloud_camera_table, t_pointcloud_camera_table):
    idx = camera_pose_indices.astype(jnp.int32)
    gather = pl.kernel(
        _gather_body,
        out_type=(
            jax.ShapeDtypeStruct((BATCH * 4,), jnp.float32),
            jax.ShapeDtypeStruct((BATCH * 3,), jnp.float32),
        ),
        mesh=plsc.VectorSubcoreMesh(core_axis_name="c", subcore_axis_name="s"),
        scratch_types=[
            pltpu.VMEM((_BPW,), jnp.int32),
            pltpu.VMEM((_QW,), jnp.int32),
            pltpu.VMEM((_TW,), jnp.int32),
            pltpu.VMEM((_QW,), jnp.float32),
            pltpu.VMEM((_TW,), jnp.float32),
            pltpu.SemaphoreType.DMA,
        ],
        compiler_params=pltpu.CompilerParams(needs_layout_passes=False,
                                             skip_device_barrier=True),
    )
    qo, to = gather(idx,
                    q_pointcloud_camera_table.reshape(-1),
                    t_pointcloud_camera_table.reshape(-1))
    return qo.reshape(BATCH, 4), to.reshape(BATCH, 3)


# speedup vs baseline: 5.4744x; 5.4654x over previous
import jax
import jax.numpy as jnp
from jax import lax
from jax.experimental import pallas as pl
from jax.experimental.pallas import tpu as pltpu
from jax.experimental.pallas import tpu_sc as plsc

NUM_POSES = 100000
BATCH = 4096

_INFO = plsc.get_sparse_core_info()
_NC = _INFO.num_cores
_NS = _INFO.num_subcores
_NW = _NC * _NS
_BPW = BATCH // _NW
_L = 16


def _gather_body(idx_hbm, qflat_hbm, tflat_hbm, qo_hbm, to_hbm,
                 iv, qiv, tiv, qcols, tcols, sem):
    wid = lax.axis_index("s") * _NC + lax.axis_index("c")
    base = wid * _BPW
    pltpu.sync_copy(idx_hbm.at[pl.ds(base, _BPW)], iv)

    for c in range(4):
        for k in range(_BPW // _L):
            qiv[pl.ds(c * _BPW + k * _L, _L)] = iv[pl.ds(k * _L, _L)] + c * NUM_POSES
    for c in range(3):
        for k in range(_BPW // _L):
            tiv[pl.ds(c * _BPW + k * _L, _L)] = iv[pl.ds(k * _L, _L)] + c * NUM_POSES

    cps = []
    for c in range(4):
        cps.append(pltpu.async_copy(
            qflat_hbm.at[qiv.at[pl.ds(c * _BPW, _BPW)]],
            qcols.at[pl.ds(c * _BPW, _BPW)], sem))
    for c in range(3):
        cps.append(pltpu.async_copy(
            tflat_hbm.at[tiv.at[pl.ds(c * _BPW, _BPW)]],
            tcols.at[pl.ds(c * _BPW, _BPW)], sem))
    for cp in cps:
        cp.wait()

    for c in range(4):
        pltpu.sync_copy(qcols.at[pl.ds(c * _BPW, _BPW)],
                        qo_hbm.at[pl.ds(c * BATCH + base, _BPW)])
    for c in range(3):
        pltpu.sync_copy(tcols.at[pl.ds(c * _BPW, _BPW)],
                        to_hbm.at[pl.ds(c * BATCH + base, _BPW)])


@jax.jit
def kernel(camera_pose_indices, q_pointcloud_camera_table, t_pointcloud_camera_table):
    idx = camera_pose_indices.astype(jnp.int32)
    gather = pl.kernel(
        _gather_body,
        out_type=(
            jax.ShapeDtypeStruct((4 * BATCH,), jnp.float32),
            jax.ShapeDtypeStruct((3 * BATCH,), jnp.float32),
        ),
        mesh=plsc.VectorSubcoreMesh(core_axis_name="c", subcore_axis_name="s"),
        scratch_types=[
            pltpu.VMEM((_BPW,), jnp.int32),
            pltpu.VMEM((_BPW * 4,), jnp.int32),
            pltpu.VMEM((_BPW * 3,), jnp.int32),
            pltpu.VMEM((_BPW * 4,), jnp.float32),
            pltpu.VMEM((_BPW * 3,), jnp.float32),
            pltpu.SemaphoreType.DMA,
        ],
        compiler_params=pltpu.CompilerParams(needs_layout_passes=False,
                                             skip_device_barrier=True),
    )
    qo, to = gather(idx,
                    q_pointcloud_camera_table.T.reshape(-1),
                    t_pointcloud_camera_table.T.reshape(-1))
    return qo.reshape(4, BATCH).T, to.reshape(3, BATCH).T


# speedup vs baseline: 5.8303x; 1.0650x over previous
import jax
import jax.numpy as jnp
from jax import lax
from jax.experimental import pallas as pl
from jax.experimental.pallas import tpu as pltpu
from jax.experimental.pallas import tpu_sc as plsc

NUM_POSES = 100000
BATCH = 4096

_INFO = plsc.get_sparse_core_info()
_NC = _INFO.num_cores
_NS = _INFO.num_subcores
_NW = _NC * _NS
_BPW = BATCH // _NW
_L = 16


def _gather_body(idx_hbm, qflat_hbm, tflat_hbm, qo_hbm, to_hbm,
                 iv, qiv, tiv, qcols, tcols, sem):
    wid = lax.axis_index("s") * _NC + lax.axis_index("c")
    base = wid * _BPW
    pltpu.sync_copy(idx_hbm.at[pl.ds(base, _BPW)], iv)

    for c in range(4):
        for k in range(_BPW // _L):
            qiv[pl.ds(c * _BPW + k * _L, _L)] = iv[pl.ds(k * _L, _L)] + c * NUM_POSES
    for c in range(3):
        for k in range(_BPW // _L):
            tiv[pl.ds(c * _BPW + k * _L, _L)] = iv[pl.ds(k * _L, _L)] + c * NUM_POSES

    cps = []
    for c in range(4):
        cps.append(pltpu.async_copy(
            qflat_hbm.at[qiv.at[pl.ds(c * _BPW, _BPW)]],
            qcols.at[pl.ds(c * _BPW, _BPW)], sem))
    for c in range(3):
        cps.append(pltpu.async_copy(
            tflat_hbm.at[tiv.at[pl.ds(c * _BPW, _BPW)]],
            tcols.at[pl.ds(c * _BPW, _BPW)], sem))
    for cp in cps:
        cp.wait()

    pltpu.sync_copy(qcols, qo_hbm.at[pl.ds(wid * (_BPW * 4), _BPW * 4)])
    pltpu.sync_copy(tcols, to_hbm.at[pl.ds(wid * (_BPW * 3), _BPW * 3)])


@jax.jit
def kernel(camera_pose_indices, q_pointcloud_camera_table, t_pointcloud_camera_table):
    idx = camera_pose_indices.astype(jnp.int32)
    gather = pl.kernel(
        _gather_body,
        out_type=(
            jax.ShapeDtypeStruct((4 * BATCH,), jnp.float32),
            jax.ShapeDtypeStruct((3 * BATCH,), jnp.float32),
        ),
        mesh=plsc.VectorSubcoreMesh(core_axis_name="c", subcore_axis_name="s"),
        scratch_types=[
            pltpu.VMEM((_BPW,), jnp.int32),
            pltpu.VMEM((_BPW * 4,), jnp.int32),
            pltpu.VMEM((_BPW * 3,), jnp.int32),
            pltpu.VMEM((_BPW * 4,), jnp.float32),
            pltpu.VMEM((_BPW * 3,), jnp.float32),
            pltpu.SemaphoreType.DMA,
        ],
        compiler_params=pltpu.CompilerParams(needs_layout_passes=False,
                                             skip_device_barrier=True),
    )
    qo, to = gather(idx,
                    q_pointcloud_camera_table.T.reshape(-1),
                    t_pointcloud_camera_table.T.reshape(-1))
    q = jnp.transpose(qo.reshape(_NW, 4, _BPW), (0, 2, 1)).reshape(BATCH, 4)
    t = jnp.transpose(to.reshape(_NW, 3, _BPW), (0, 2, 1)).reshape(BATCH, 3)
    return q, t


# speedup vs baseline: 5.8520x; 1.0037x over previous
import jax
import jax.numpy as jnp
from jax import lax
from jax.experimental import pallas as pl
from jax.experimental.pallas import tpu as pltpu
from jax.experimental.pallas import tpu_sc as plsc

NUM_POSES = 100000
BATCH = 4096

_INFO = plsc.get_sparse_core_info()
_NC = _INFO.num_cores
_NS = _INFO.num_subcores
_NW = _NC * _NS
_BPW = BATCH // _NW
_L = 16


def _gather_body(idx_hbm, qflat_hbm, tflat_hbm, qo_hbm, to_hbm,
                 iv, qcols, tcols, sem):
    wid = lax.axis_index("s") * _NC + lax.axis_index("c")
    base = wid * _BPW
    pltpu.sync_copy(idx_hbm.at[pl.ds(base, _BPW)], iv)

    cps = []
    for c in range(4):
        cps.append(pltpu.async_copy(
            qflat_hbm.at[pl.ds(c * NUM_POSES, NUM_POSES)].at[iv],
            qcols.at[pl.ds(c * _BPW, _BPW)], sem))
    for c in range(3):
        cps.append(pltpu.async_copy(
            tflat_hbm.at[pl.ds(c * NUM_POSES, NUM_POSES)].at[iv],
            tcols.at[pl.ds(c * _BPW, _BPW)], sem))
    for cp in cps:
        cp.wait()

    pltpu.sync_copy(qcols, qo_hbm.at[pl.ds(wid * (_BPW * 4), _BPW * 4)])
    pltpu.sync_copy(tcols, to_hbm.at[pl.ds(wid * (_BPW * 3), _BPW * 3)])


@jax.jit
def kernel(camera_pose_indices, q_pointcloud_camera_table, t_pointcloud_camera_table):
    idx = camera_pose_indices.astype(jnp.int32)
    gather = pl.kernel(
        _gather_body,
        out_type=(
            jax.ShapeDtypeStruct((4 * BATCH,), jnp.float32),
            jax.ShapeDtypeStruct((3 * BATCH,), jnp.float32),
        ),
        mesh=plsc.VectorSubcoreMesh(core_axis_name="c", subcore_axis_name="s"),
        scratch_types=[
            pltpu.VMEM((_BPW,), jnp.int32),
            pltpu.VMEM((_BPW * 4,), jnp.float32),
            pltpu.VMEM((_BPW * 3,), jnp.float32),
            pltpu.SemaphoreType.DMA,
        ],
        compiler_params=pltpu.CompilerParams(needs_layout_passes=False,
                                             skip_device_barrier=True),
    )
    qo, to = gather(idx,
                    q_pointcloud_camera_table.T.reshape(-1),
                    t_pointcloud_camera_table.T.reshape(-1))
    q = jnp.transpose(qo.reshape(_NW, 4, _BPW), (0, 2, 1)).reshape(BATCH, 4)
    t = jnp.transpose(to.reshape(_NW, 3, _BPW), (0, 2, 1)).reshape(BATCH, 3)
    return q, t


# speedup vs baseline: 5.9393x; 1.0149x over previous
import jax
import jax.numpy as jnp
from jax import lax
from jax.experimental import pallas as pl
from jax.experimental.pallas import tpu as pltpu
from jax.experimental.pallas import tpu_sc as plsc

NUM_POSES = 100000
BATCH = 4096

_INFO = plsc.get_sparse_core_info()
_NC = _INFO.num_cores
_NS = _INFO.num_subcores
_NW = _NC * _NS
_BPW = BATCH // _NW
_L = 16


def _gather_body(idx_hbm, flat_hbm, qo_hbm, to_hbm,
                 iv, qcols, tcols, sem):
    wid = lax.axis_index("s") * _NC + lax.axis_index("c")
    base = wid * _BPW
    pltpu.sync_copy(idx_hbm.at[pl.ds(base, _BPW)], iv)

    cps = []
    for c in range(4):
        cps.append(pltpu.async_copy(
            flat_hbm.at[pl.ds(c * NUM_POSES, NUM_POSES)].at[iv],
            qcols.at[pl.ds(c * _BPW, _BPW)], sem))
    for c in range(3):
        cps.append(pltpu.async_copy(
            flat_hbm.at[pl.ds((4 + c) * NUM_POSES, NUM_POSES)].at[iv],
            tcols.at[pl.ds(c * _BPW, _BPW)], sem))
    for cp in cps:
        cp.wait()

    pltpu.sync_copy(qcols, qo_hbm.at[pl.ds(wid * (_BPW * 4), _BPW * 4)])
    pltpu.sync_copy(tcols, to_hbm.at[pl.ds(wid * (_BPW * 3), _BPW * 3)])


@jax.jit
def kernel(camera_pose_indices, q_pointcloud_camera_table, t_pointcloud_camera_table):
    idx = camera_pose_indices.astype(jnp.int32)
    gather = pl.kernel(
        _gather_body,
        out_type=(
            jax.ShapeDtypeStruct((4 * BATCH,), jnp.float32),
            jax.ShapeDtypeStruct((3 * BATCH,), jnp.float32),
        ),
        mesh=plsc.VectorSubcoreMesh(core_axis_name="c", subcore_axis_name="s"),
        scratch_types=[
            pltpu.VMEM((_BPW,), jnp.int32),
            pltpu.VMEM((_BPW * 4,), jnp.float32),
            pltpu.VMEM((_BPW * 3,), jnp.float32),
            pltpu.SemaphoreType.DMA,
        ],
        compiler_params=pltpu.CompilerParams(needs_layout_passes=False,
                                             skip_device_barrier=True),
    )
    flat = jnp.concatenate([q_pointcloud_camera_table.T.reshape(-1),
                            t_pointcloud_camera_table.T.reshape(-1)])
    qo, to = gather(idx, flat)
    q = jnp.transpose(qo.reshape(_NW, 4, _BPW), (0, 2, 1)).reshape(BATCH, 4)
    t = jnp.transpose(to.reshape(_NW, 3, _BPW), (0, 2, 1)).reshape(BATCH, 3)
    return q, t


# speedup vs baseline: 6.3389x; 1.0673x over previous
import jax
import jax.numpy as jnp
from jax import lax
from jax.experimental import pallas as pl
from jax.experimental.pallas import tpu as pltpu
from jax.experimental.pallas import tpu_sc as plsc

NUM_POSES = 100000
BATCH = 4096

_INFO = plsc.get_sparse_core_info()
_NC = _INFO.num_cores
_NS = _INFO.num_subcores
_NW = _NC * _NS
_BPW = BATCH // _NW
_L = 16


def _gather_body(idx_hbm, flat_hbm, qo_hbm, to_hbm,
                 iv, qcols, tcols, sem):
    wid = lax.axis_index("s") * _NC + lax.axis_index("c")
    base = wid * _BPW
    pltpu.sync_copy(idx_hbm.at[pl.ds(base, _BPW)], iv)

    qcps = [pltpu.async_copy(
        flat_hbm.at[pl.ds(c * NUM_POSES, NUM_POSES)].at[iv],
        qcols.at[pl.ds(c * _BPW, _BPW)], sem) for c in range(4)]
    tcps = [pltpu.async_copy(
        flat_hbm.at[pl.ds((4 + c) * NUM_POSES, NUM_POSES)].at[iv],
        tcols.at[pl.ds(c * _BPW, _BPW)], sem) for c in range(3)]
    for cp in qcps:
        cp.wait()
    pltpu.sync_copy(qcols, qo_hbm.at[pl.ds(wid * (_BPW * 4), _BPW * 4)])
    for cp in tcps:
        cp.wait()
    pltpu.sync_copy(tcols, to_hbm.at[pl.ds(wid * (_BPW * 4), _BPW * 3)])


@jax.jit
def kernel(camera_pose_indices, q_pointcloud_camera_table, t_pointcloud_camera_table):
    idx = camera_pose_indices.astype(jnp.int32)
    gather = pl.kernel(
        _gather_body,
        out_type=(
            jax.ShapeDtypeStruct((4 * BATCH,), jnp.float32),
            jax.ShapeDtypeStruct((4 * BATCH,), jnp.float32),
        ),
        mesh=plsc.VectorSubcoreMesh(core_axis_name="c", subcore_axis_name="s"),
        scratch_types=[
            pltpu.VMEM((_BPW,), jnp.int32),
            pltpu.VMEM((_BPW * 4,), jnp.float32),
            pltpu.VMEM((_BPW * 3,), jnp.float32),
            pltpu.SemaphoreType.DMA,
        ],
        compiler_params=pltpu.CompilerParams(needs_layout_passes=False,
                                             skip_device_barrier=True),
    )
    flat = jnp.concatenate([q_pointcloud_camera_table.T.reshape(-1),
                            t_pointcloud_camera_table.T.reshape(-1)])
    qo, to = gather(idx, flat)
    q = jnp.transpose(qo.reshape(_NW, 4, _BPW), (0, 2, 1)).reshape(BATCH, 4)
    t = jnp.transpose(to.reshape(_NW, 4, _BPW), (0, 2, 1)).reshape(BATCH, 4)[:, :3]
    return q, t
